# Initial kernel scaffold; baseline (speedup 1.0000x reference)
import jax
import jax.numpy as jnp
from jax.experimental import pallas as pl


def kernel(x, edge_attr, edge_index):
    raise NotImplementedError("write your pallas kernel here")


# speedup vs baseline: 37.5815x; 37.5815x over previous
import functools

import jax
import jax.numpy as jnp
from jax import lax
from jax.experimental import pallas as pl
from jax.experimental.pallas import tpu as pltpu
from jax.experimental.pallas import tpu_sc as plsc

_NC = 2
_NS = 16
_NW = _NC * _NS
_CH = 256


def _sc_seq_update(xw, eaw, src, dst, n_nodes, lpw):
    E = src.shape[0]
    nch = E // _CH
    assert E % _CH == 0

    mesh = plsc.VectorSubcoreMesh(core_axis_name="c", subcore_axis_name="s")

    @functools.partial(
        pl.kernel,
        mesh=mesh,
        compiler_params=pltpu.CompilerParams(
            use_tc_tiling_on_sc=False, needs_layout_passes=False),
        out_type=jax.ShapeDtypeStruct((_NW, n_nodes * lpw), jnp.float32),
        scratch_types=[
            pltpu.VMEM((n_nodes * lpw,), jnp.float32),
            pltpu.VMEM((2, _CH * lpw), jnp.float32),
            pltpu.VMEM((2, _CH), jnp.int32),
            pltpu.VMEM((2, _CH), jnp.int32),
            pltpu.SemaphoreType.DMA((2,)),
        ],
    )
    def run(x_hbm, ea_hbm, src_hbm, dst_hbm, out_hbm, temp, eabuf, sbuf, dbuf, sem):
        wid = lax.axis_index("s") * _NC + lax.axis_index("c")

        pltpu.sync_copy(x_hbm.at[wid], temp)

        lane = lax.broadcasted_iota(jnp.int32, (16,), 0)
        msk = lane < lpw
        col = jnp.where(msk, lane, 0)

        def dmas(g, b):
            return (
                pltpu.make_async_copy(
                    ea_hbm.at[wid, pl.ds(g * _CH * lpw, _CH * lpw)],
                    eabuf.at[b], sem.at[b]),
                pltpu.make_async_copy(
                    src_hbm.at[pl.ds(g * _CH, _CH)], sbuf.at[b], sem.at[b]),
                pltpu.make_async_copy(
                    dst_hbm.at[pl.ds(g * _CH, _CH)], dbuf.at[b], sem.at[b]),
            )

        def issue(g, b):
            for d in dmas(g, b):
                d.start()

        def wait(g, b):
            for d in dmas(g, b):
                d.wait()

        def process(b):
            def edge(k, ebase):
                kk = jnp.full((16,), k, jnp.int32)
                s = plsc.load_gather(sbuf.at[b], [kk])
                d = plsc.load_gather(dbuf.at[b], [kk])
                row = plsc.load_gather(temp, [s + col], mask=msk)
                e = plsc.load_gather(eabuf.at[b], [ebase + col], mask=msk)
                plsc.store_scatter(temp, [d + col], row - e, mask=msk)
                return ebase + lpw
            lax.fori_loop(0, _CH, edge, jnp.zeros((16,), jnp.int32))

        issue(0, 0)

        def pair(p, carry):
            g0 = 2 * p

            @pl.when(g0 + 1 < nch)
            def _():
                issue(g0 + 1, 1)

            wait(g0, 0)
            process(0)

            @pl.when(g0 + 2 < nch)
            def _():
                issue(g0 + 2, 0)

            @pl.when(g0 + 1 < nch)
            def _():
                wait(g0 + 1, 1)
                process(1)

            return carry

        lax.fori_loop(0, (nch + 1) // 2, pair, 0)

        pltpu.sync_copy(temp, out_hbm.at[wid])

    return run(xw, eaw, src, dst)


def kernel(x, edge_attr, edge_index):
    T, N, D = x.shape
    E = edge_attr.shape[1]
    ncols = (T - 1) * D
    lpw = ncols // _NW
    assert ncols % _NW == 0

    xw = (x[:-1].transpose(1, 0, 2).reshape(N, _NW, lpw)
          .transpose(1, 0, 2).reshape(_NW, N * lpw))
    eaw = (edge_attr.transpose(1, 0, 2).reshape(E, _NW, lpw)
           .transpose(1, 0, 2).reshape(_NW, E * lpw))
    src = edge_index[0].astype(jnp.int32) * lpw
    dst = edge_index[1].astype(jnp.int32) * lpw

    out = _sc_seq_update(xw, eaw, src, dst, N, lpw)
    updated = (out.reshape(_NW, N, lpw).transpose(1, 0, 2).reshape(N, T - 1, D)
               .transpose(1, 0, 2))
    return jnp.concatenate([x[:1], updated], axis=0)
